# baseline (speedup 1.0000x reference)
import functools
import jax
import jax.numpy as jnp
from jax import lax
from jax.experimental import pallas as pl
from jax.experimental.pallas import tpu as pltpu
from jax.experimental.pallas import tpu_sc as plsc

N_IN = 10
N_OUT = 64
M_ROWS = 16384
NVARS = 2 ** N_IN - 2

NCB = 4
NRB = 8
ROWS_PER_TILE = M_ROWS // NRB
GROUPS = ROWS_PER_TILE // 16
CW = N_OUT // NCB


def _batcher_pairs(n):
    pairs = []
    p2 = 1
    while p2 < n:
        p2 *= 2

    def compare(a, b):
        if a < n and b < n:
            pairs.append((a, b))

    def merge(lo, hi, r):
        step = r * 2
        if step < hi - lo:
            merge(lo, hi, step)
            merge(lo + r, hi, step)
            for i in range(lo + r, hi - r, step):
                compare(i, i + r)
        else:
            compare(lo, lo + r)

    def sort(lo, hi):
        if hi - lo >= 1:
            mid = lo + (hi - lo) // 2
            sort(lo, mid)
            sort(mid + 1, hi)
            merge(lo, hi, 1)

    sort(0, p2 - 1)
    return pairs


_PAIRS = _batcher_pairs(N_IN)


def _round_bf16(x):
    u = plsc.bitcast(x, jnp.uint32)
    lsb = jnp.bitwise_and(
        lax.shift_right_logical(u, jnp.full((16,), 16, jnp.uint32)),
        jnp.full((16,), 1, jnp.uint32))
    u = u + lsb + jnp.full((16,), 0x7FFF, jnp.uint32)
    u = jnp.bitwise_and(u, jnp.full((16,), 0xFFFF0000, jnp.uint32))
    return plsc.bitcast(u, jnp.float32)

_mesh = plsc.VectorSubcoreMesh(core_axis_name="c", subcore_axis_name="s")


@functools.partial(
    pl.kernel,
    out_type=jax.ShapeDtypeStruct((M_ROWS, N_OUT), jnp.float32),
    mesh=_mesh,
    scratch_types=[
        pltpu.VMEM(((NVARS + 2) * CW,), jnp.float32),
        pltpu.VMEM(((NVARS + 2) * CW,), jnp.float32),
        pltpu.VMEM((NVARS * CW,), jnp.float32),
        pltpu.VMEM((ROWS_PER_TILE * N_IN,), jnp.float32),
        pltpu.VMEM((ROWS_PER_TILE, CW), jnp.float32),
        pltpu.SemaphoreType.DMA,
    ],
    compiler_params=pltpu.CompilerParams(needs_layout_passes=False,
                                         use_tc_tiling_on_sc=False),
)
def _choquet_sc(in_hbm, vars_hbm, out_hbm, T, Tc, chi, xin, ob, sem):
    wid = lax.axis_index("c") * 16 + lax.axis_index("s")
    rb = wid // NCB
    cb = lax.rem(wid, NCB)
    r0 = rb * ROWS_PER_TILE

    in_copy = pltpu.make_async_copy(
        in_hbm.at[pl.ds(r0 * N_IN, ROWS_PER_TILE * N_IN)], xin, sem)
    in_copy.start()
    pltpu.sync_copy(vars_hbm.at[cb], chi)

    iota = lax.iota(jnp.int32, 16)
    zeros = jnp.zeros((16,), jnp.float32)
    ones = jnp.ones((16,), jnp.float32)
    fifteen = jnp.full((16,), 15, jnp.int32)

    T[pl.ds(0, 16)] = zeros
    Tc[pl.ds((NVARS + 1) * CW, 16)] = ones

    def dp_body(s, carry):
        addrs = []
        for b in range(N_IN):
            bit = lax.rem(lax.shift_right_logical(s, b), 2)
            addrs.append(jnp.where(bit == 1, (s - (1 << b)) * CW, 0))
        m = T[pl.ds(addrs[0], 16)]
        for b in range(1, N_IN):
            m = jnp.maximum(m, T[pl.ds(addrs[b], 16)])
        row = m + jnp.abs(chi[pl.ds((s - 1) * CW, 16)])
        T[pl.ds(s * CW, 16)] = row
        Tc[pl.ds(s * CW, 16)] = _round_bf16(jnp.minimum(row, 1.0))
        return carry

    lax.fori_loop(1, NVARS + 1, dp_body, 0)

    in_copy.wait()

    i10 = iota * N_IN

    @plsc.parallel_loop(0, GROUPS, 1, unroll=4)
    def row_group(g):
        rvec = iota + g * 16
        base10 = i10 + g * (16 * N_IN)
        xs = [plsc.load_gather(xin, [base10 + f]) for f in range(N_IN)]
        ps = [jnp.full((16,), 1 << f, jnp.int32) for f in range(N_IN)]
        for a, b in _PAIRS:
            swap = xs[a] < xs[b]
            hi = jnp.maximum(xs[a], xs[b])
            lo = jnp.minimum(xs[a], xs[b])
            pa = jnp.where(swap, ps[b], ps[a])
            pb = jnp.where(swap, ps[a], ps[b])
            xs[a], xs[b] = hi, lo
            ps[a], ps[b] = pa, pb
        cums = [ps[0]]
        for r in range(1, N_IN):
            cums.append(cums[-1] + ps[r])
        a16 = [lax.shift_left(c, 4) for c in cums]
        ds = [_round_bf16(xs[r] - xs[r + 1]) for r in range(N_IN - 1)]
        ds.append(_round_bf16(xs[N_IN - 1]))
        rows16 = rvec
        for j in range(CW):
            colv = jnp.bitwise_and(iota + j, fifteen)
            acc = ds[N_IN - 1] + ds[0] * plsc.load_gather(Tc, [a16[0] + colv])
            for r in range(1, N_IN - 1):
                acc = acc + ds[r] * plsc.load_gather(Tc, [a16[r] + colv])
            plsc.store_scatter(ob, [rows16, colv], acc)

    pltpu.sync_copy(ob, out_hbm.at[pl.ds(r0, ROWS_PER_TILE),
                                   pl.ds(cb * CW, CW)])


def kernel(inputs, vars):
    in_flat = inputs.reshape(-1)
    vars_blk = vars.reshape(NVARS, NCB, CW).transpose(1, 0, 2).reshape(NCB, -1)
    return _choquet_sc(in_flat, vars_blk)


# speedup vs baseline: 1.5635x; 1.5635x over previous
import functools
import jax
import jax.numpy as jnp
from jax import lax
from jax.experimental import pallas as pl
from jax.experimental.pallas import tpu as pltpu
from jax.experimental.pallas import tpu_sc as plsc

N_IN = 10
N_OUT = 64
M_ROWS = 16384
NVARS = 2 ** N_IN - 2

NCB = 4
NRB = 8
ROWS_PER_TILE = M_ROWS // NRB
GROUPS = ROWS_PER_TILE // 16
CW = N_OUT // NCB


def _batcher_pairs(n):
    pairs = []
    p2 = 1
    while p2 < n:
        p2 *= 2

    def compare(a, b):
        if a < n and b < n:
            pairs.append((a, b))

    def merge(lo, hi, r):
        step = r * 2
        if step < hi - lo:
            merge(lo, hi, step)
            merge(lo + r, hi, step)
            for i in range(lo + r, hi - r, step):
                compare(i, i + r)
        else:
            compare(lo, lo + r)

    def sort(lo, hi):
        if hi - lo >= 1:
            mid = lo + (hi - lo) // 2
            sort(lo, mid)
            sort(mid + 1, hi)
            merge(lo, hi, 1)

    sort(0, p2 - 1)
    return pairs


_PAIRS = _batcher_pairs(N_IN)


def _round_bf16(x):
    u = plsc.bitcast(x, jnp.uint32)
    lsb = jnp.bitwise_and(
        lax.shift_right_logical(u, jnp.full((16,), 16, jnp.uint32)),
        jnp.full((16,), 1, jnp.uint32))
    u = u + lsb + jnp.full((16,), 0x7FFF, jnp.uint32)
    u = jnp.bitwise_and(u, jnp.full((16,), 0xFFFF0000, jnp.uint32))
    return plsc.bitcast(u, jnp.float32)

_mesh = plsc.VectorSubcoreMesh(core_axis_name="c", subcore_axis_name="s")


@functools.partial(
    pl.kernel,
    out_type=jax.ShapeDtypeStruct((M_ROWS, N_OUT), jnp.float32),
    mesh=_mesh,
    scratch_types=[
        pltpu.VMEM(((NVARS + 2) * CW,), jnp.float32),
        pltpu.VMEM(((NVARS + 2) * CW,), jnp.float32),
        pltpu.VMEM((NVARS * CW,), jnp.float32),
        pltpu.VMEM((ROWS_PER_TILE * N_IN,), jnp.float32),
        pltpu.VMEM((ROWS_PER_TILE, CW), jnp.float32),
        pltpu.SemaphoreType.DMA,
    ],
    compiler_params=pltpu.CompilerParams(needs_layout_passes=False,
                                         use_tc_tiling_on_sc=False),
)
def _choquet_sc(in_hbm, vars_hbm, out_hbm, T, Tc, chi, xin, ob, sem):
    wid = lax.axis_index("c") * 16 + lax.axis_index("s")
    rb = wid // NCB
    cb = lax.rem(wid, NCB)
    r0 = rb * ROWS_PER_TILE

    in_copy = pltpu.make_async_copy(
        in_hbm.at[pl.ds(r0 * N_IN, ROWS_PER_TILE * N_IN)], xin, sem)
    in_copy.start()
    pltpu.sync_copy(vars_hbm.at[cb], chi)

    iota = lax.iota(jnp.int32, 16)
    zeros = jnp.zeros((16,), jnp.float32)
    ones = jnp.ones((16,), jnp.float32)
    fifteen = jnp.full((16,), 15, jnp.int32)

    T[pl.ds(0, 16)] = zeros
    Tc[pl.ds((NVARS + 1) * CW, 16)] = ones

    def dp_body(s, carry):
        addrs = []
        for b in range(N_IN):
            bit = lax.rem(lax.shift_right_logical(s, b), 2)
            addrs.append(jnp.where(bit == 1, (s - (1 << b)) * CW, 0))
        m = T[pl.ds(addrs[0], 16)]
        for b in range(1, N_IN):
            m = jnp.maximum(m, T[pl.ds(addrs[b], 16)])
        row = m + jnp.abs(chi[pl.ds((s - 1) * CW, 16)])
        T[pl.ds(s * CW, 16)] = row
        Tc[pl.ds(s * CW, 16)] = _round_bf16(jnp.minimum(row, 1.0))
        return carry

    lax.fori_loop(1, NVARS + 1, dp_body, 0)

    in_copy.wait()

    i10 = iota * N_IN

    @plsc.parallel_loop(0, GROUPS, 1, unroll=2)
    def row_group(g):
        rvec = iota + g * 16
        base10 = i10 + g * (16 * N_IN)
        xs = [plsc.load_gather(xin, [base10 + f]) for f in range(N_IN)]
        ps = [jnp.full((16,), 1 << f, jnp.int32) for f in range(N_IN)]
        for a, b in _PAIRS:
            swap = xs[a] < xs[b]
            hi = jnp.maximum(xs[a], xs[b])
            lo = jnp.minimum(xs[a], xs[b])
            pa = jnp.where(swap, ps[b], ps[a])
            pb = jnp.where(swap, ps[a], ps[b])
            xs[a], xs[b] = hi, lo
            ps[a], ps[b] = pa, pb
        cums = [ps[0]]
        for r in range(1, N_IN):
            cums.append(cums[-1] + ps[r])
        a16 = [lax.shift_left(c, 4) for c in cums]
        ds = [_round_bf16(xs[r] - xs[r + 1]) for r in range(N_IN - 1)]
        ds.append(_round_bf16(xs[N_IN - 1]))
        rows16 = rvec
        for j in range(CW):
            colv = jnp.bitwise_and(iota + j, fifteen)
            acc = ds[N_IN - 1] + ds[0] * plsc.load_gather(Tc, [a16[0] + colv])
            for r in range(1, N_IN - 1):
                acc = acc + ds[r] * plsc.load_gather(Tc, [a16[r] + colv])
            plsc.store_scatter(ob, [rows16, colv], acc)

    pltpu.sync_copy(ob, out_hbm.at[pl.ds(r0, ROWS_PER_TILE),
                                   pl.ds(cb * CW, CW)])


def kernel(inputs, vars):
    in_flat = inputs.reshape(-1)
    vars_blk = vars.reshape(NVARS, NCB, CW).transpose(1, 0, 2).reshape(NCB, -1)
    return _choquet_sc(in_flat, vars_blk)


# speedup vs baseline: 1.5655x; 1.0013x over previous
import functools
import jax
import jax.numpy as jnp
from jax import lax
from jax.experimental import pallas as pl
from jax.experimental.pallas import tpu as pltpu
from jax.experimental.pallas import tpu_sc as plsc

N_IN = 10
N_OUT = 64
M_ROWS = 16384
NVARS = 2 ** N_IN - 2

NCB = 4
NRB = 8
ROWS_PER_TILE = M_ROWS // NRB
GROUPS = ROWS_PER_TILE // 16
CW = N_OUT // NCB


def _batcher_pairs(n):
    pairs = []
    p2 = 1
    while p2 < n:
        p2 *= 2

    def compare(a, b):
        if a < n and b < n:
            pairs.append((a, b))

    def merge(lo, hi, r):
        step = r * 2
        if step < hi - lo:
            merge(lo, hi, step)
            merge(lo + r, hi, step)
            for i in range(lo + r, hi - r, step):
                compare(i, i + r)
        else:
            compare(lo, lo + r)

    def sort(lo, hi):
        if hi - lo >= 1:
            mid = lo + (hi - lo) // 2
            sort(lo, mid)
            sort(mid + 1, hi)
            merge(lo, hi, 1)

    sort(0, p2 - 1)
    return pairs


_PAIRS = _batcher_pairs(N_IN)


def _round_bf16(x):
    u = plsc.bitcast(x, jnp.uint32)
    lsb = jnp.bitwise_and(
        lax.shift_right_logical(u, jnp.full((16,), 16, jnp.uint32)),
        jnp.full((16,), 1, jnp.uint32))
    u = u + lsb + jnp.full((16,), 0x7FFF, jnp.uint32)
    u = jnp.bitwise_and(u, jnp.full((16,), 0xFFFF0000, jnp.uint32))
    return plsc.bitcast(u, jnp.float32)

_mesh = plsc.VectorSubcoreMesh(core_axis_name="c", subcore_axis_name="s")


@functools.partial(
    pl.kernel,
    out_type=jax.ShapeDtypeStruct((M_ROWS, N_OUT), jnp.float32),
    mesh=_mesh,
    scratch_types=[
        pltpu.VMEM(((NVARS + 2) * CW,), jnp.float32),
        pltpu.VMEM(((NVARS + 2) * CW,), jnp.float32),
        pltpu.VMEM((NVARS * CW,), jnp.float32),
        pltpu.VMEM((ROWS_PER_TILE * N_IN,), jnp.float32),
        pltpu.VMEM((ROWS_PER_TILE, CW), jnp.float32),
        pltpu.SemaphoreType.DMA,
    ],
    compiler_params=pltpu.CompilerParams(needs_layout_passes=False,
                                         use_tc_tiling_on_sc=False),
)
def _choquet_sc(in_hbm, vars_hbm, out_hbm, T, Tc, chi, xin, ob, sem):
    wid = lax.axis_index("c") * 16 + lax.axis_index("s")
    rb = wid // NCB
    cb = lax.rem(wid, NCB)
    r0 = rb * ROWS_PER_TILE

    in_copy = pltpu.make_async_copy(
        in_hbm.at[pl.ds(r0 * N_IN, ROWS_PER_TILE * N_IN)], xin, sem)
    in_copy.start()
    pltpu.sync_copy(vars_hbm.at[cb], chi)

    iota = lax.iota(jnp.int32, 16)
    zeros = jnp.zeros((16,), jnp.float32)
    ones = jnp.ones((16,), jnp.float32)
    fifteen = jnp.full((16,), 15, jnp.int32)

    T[pl.ds(0, 16)] = zeros
    Tc[pl.ds((NVARS + 1) * CW, 16)] = ones

    def dp_one(s):
        addrs = []
        for b in range(N_IN):
            bit = lax.rem(lax.shift_right_logical(s, b), 2)
            addrs.append(jnp.where(bit == 1, (s - (1 << b)) * CW, 0))
        ls = [T[pl.ds(a, 16)] for a in addrs]
        while len(ls) > 1:
            ls = [jnp.maximum(ls[i], ls[i + 1])
                  for i in range(0, len(ls) - 1, 2)] + \
                 (ls[-1:] if len(ls) % 2 else [])
        row = ls[0] + jnp.abs(chi[pl.ds((s - 1) * CW, 16)])
        T[pl.ds(s * CW, 16)] = row
        Tc[pl.ds(s * CW, 16)] = _round_bf16(jnp.minimum(row, 1.0))

    def dp_pair(i, carry):
        dp_one(2 * i + 1)
        dp_one(2 * i + 2)
        return carry

    lax.fori_loop(0, NVARS // 2, dp_pair, 0)

    in_copy.wait()

    i10 = iota * N_IN

    @plsc.parallel_loop(0, GROUPS, 1, unroll=2)
    def row_group(g):
        rvec = iota + g * 16
        base10 = i10 + g * (16 * N_IN)
        xs = [plsc.load_gather(xin, [base10 + f]) for f in range(N_IN)]
        ps = [jnp.full((16,), 1 << f, jnp.int32) for f in range(N_IN)]
        for a, b in _PAIRS:
            swap = xs[a] < xs[b]
            hi = jnp.maximum(xs[a], xs[b])
            lo = jnp.minimum(xs[a], xs[b])
            pa = jnp.where(swap, ps[b], ps[a])
            pb = jnp.where(swap, ps[a], ps[b])
            xs[a], xs[b] = hi, lo
            ps[a], ps[b] = pa, pb
        cums = [ps[0]]
        for r in range(1, N_IN):
            cums.append(cums[-1] + ps[r])
        a16 = [lax.shift_left(c, 4) for c in cums]
        ds = [_round_bf16(xs[r] - xs[r + 1]) for r in range(N_IN - 1)]
        ds.append(_round_bf16(xs[N_IN - 1]))
        rows16 = rvec
        for j in range(CW):
            colv = jnp.bitwise_and(iota + j, fifteen)
            acc = ds[N_IN - 1] + ds[0] * plsc.load_gather(Tc, [a16[0] + colv])
            for r in range(1, N_IN - 1):
                acc = acc + ds[r] * plsc.load_gather(Tc, [a16[r] + colv])
            plsc.store_scatter(ob, [rows16, colv], acc)

    pltpu.sync_copy(ob, out_hbm.at[pl.ds(r0, ROWS_PER_TILE),
                                   pl.ds(cb * CW, CW)])


def kernel(inputs, vars):
    in_flat = inputs.reshape(-1)
    vars_blk = vars.reshape(NVARS, NCB, CW).transpose(1, 0, 2).reshape(NCB, -1)
    return _choquet_sc(in_flat, vars_blk)
